# baseline (speedup 1.0000x reference)
import functools

import jax
import jax.numpy as jnp
import numpy as np
from jax import lax
from jax.experimental import pallas as pl
from jax.experimental.pallas import tpu as pltpu
from jax.experimental.pallas import tpu_sc as plsc

_N = 10000
_H = 16
_C = 256
_K = 15
_CPG = 8
_G = _C // _CPG
_CR = 64
_SIGMA = 1.2
_BN_EPS = 1e-5

_DI = 256
_B = _N * _H
_NW = 32
_CHUNK = 128
_NITER = 40
_NCH = _NW * _NITER
_BPAD = _NCH * _CHUNK
_BOUT = 1300 * _CHUNK

_BN = 200
_NB = _N // _BN
_R = _BN * _H
_PR = _R // _CHUNK


def _kp_const():
    rng = np.random.RandomState(42)
    pts = rng.randn(_K, 3)
    pts = pts / (np.linalg.norm(pts, axis=1, keepdims=True) + 1e-9)
    r = rng.rand(_K, 1) ** (1.0 / 3.0)
    pts = pts * r * 1.2
    pts[0, :] = 0.0
    return pts.astype(np.float32)


def _sc_gather(table, idx):
    mesh = plsc.VectorSubcoreMesh(core_axis_name="c", subcore_axis_name="s")
    ipw = _NITER * _CHUNK

    @functools.partial(
        pl.kernel,
        mesh=mesh,
        out_type=jax.ShapeDtypeStruct((_BOUT, _DI), jnp.int32),
        scratch_types=[
            pltpu.VMEM((ipw,), jnp.int32),
            pltpu.VMEM((_CHUNK, _DI), jnp.int32),
            pltpu.VMEM((_CHUNK, _DI), jnp.int32),
            pltpu.SemaphoreType.DMA,
            pltpu.SemaphoreType.DMA,
            pltpu.SemaphoreType.DMA,
        ],
    )
    def k(table_hbm, idx_hbm, out_f,
          idx_v, rows0, rows1, gsem, osem0, osem1):
        wid = lax.axis_index("s") * 2 + lax.axis_index("c")
        base_chunk = wid * _NITER

        rows_v = (rows0, rows1)
        osem = (osem0, osem1)

        pltpu.sync_copy(idx_hbm.at[pl.ds(base_chunk * _CHUNK, ipw)], idx_v)

        pend_o = [None, None]
        for i in range(_NITER):
            b = i % 2
            ci = base_chunk + i
            if pend_o[b] is not None:
                pend_o[b].wait()
            pltpu.async_copy(
                table_hbm.at[idx_v.at[pl.ds(i * _CHUNK, _CHUNK)]],
                rows_v[b], gsem).wait()
            pend_o[b] = pltpu.async_copy(
                rows_v[b],
                out_f.at[pl.ds(ci * _CHUNK, _CHUNK)],
                osem[b])
        pend_o[0].wait()
        pend_o[1].wait()

    return k(table, idx)


def _stats_body(x_ref, w1_ref, b1_ref, s1_ref, s2_ref):
    h = jnp.dot(x_ref[...], w1_ref[...],
                preferred_element_type=jnp.float32) + b1_ref[...]

    @pl.when(pl.program_id(0) == 0)
    def _():
        s1_ref[...] = jnp.zeros_like(s1_ref)
        s2_ref[...] = jnp.zeros_like(s2_ref)

    s1_ref[...] += jnp.sum(h, axis=0, keepdims=True)
    s2_ref[...] += jnp.sum(h * h, axis=0, keepdims=True)


def _tc_stats(s_feats, W1, b1r):
    return pl.pallas_call(
        _stats_body,
        grid=(_NB,),
        in_specs=[
            pl.BlockSpec((_BN, _C), lambda i: (i, 0)),
            pl.BlockSpec((_C, _CR), lambda i: (0, 0)),
            pl.BlockSpec((1, _CR), lambda i: (0, 0)),
        ],
        out_specs=[
            pl.BlockSpec((1, _CR), lambda i: (0, 0)),
            pl.BlockSpec((1, _CR), lambda i: (0, 0)),
        ],
        out_shape=[
            jax.ShapeDtypeStruct((1, _CR), jnp.float32),
            jax.ShapeDtypeStruct((1, _CR), jnp.float32),
        ],
        compiler_params=pltpu.CompilerParams(
            dimension_semantics=("arbitrary",)),
    )(s_feats, W1, b1r)


def _rep_rows(a, m):
    return jnp.broadcast_to(a[:, None, :], (_BN, _H, m)).reshape(_R, m)


def _main_body(x_ref, q_ref, g_ref,
               s1_ref, s2_ref, w1_ref, b1_ref,
               gam_ref, bet_ref, w2_ref, b2_ref, kp_ref, o_ref):
    x = x_ref[...]
    h = jnp.dot(x, w1_ref[...], preferred_element_type=jnp.float32)
    h = h + b1_ref[...]
    mean = s1_ref[...] * (1.0 / _N)
    var = s2_ref[...] * (1.0 / _N) - mean * mean
    inv = lax.rsqrt(var + _BN_EPS)
    h = (h - mean) * (inv * gam_ref[...]) + bet_ref[...]
    h = jnp.where(h >= 0.0, h, 0.1 * h)
    cw = jnp.dot(h, w2_ref[...], preferred_element_type=jnp.float32)
    cw = cw + b2_ref[...]

    gw = g_ref[...]
    feats = jax.lax.bitcast_convert_type(
        jax.lax.shift_left(gw, 16), jnp.float32)
    q = q_ref[...]
    m5 = kp_ref[...]

    mw = gw[:, 0:6]
    pos = jax.lax.bitcast_convert_type(
        jax.lax.bitwise_and(mw, jnp.int32(-65536)), jnp.float32)
    pxf = pos[:, 0:1] + pos[:, 1:2]
    pyf = pos[:, 2:3] + pos[:, 3:4]
    pzf = pos[:, 4:5] + pos[:, 5:6]
    ax = pxf - _rep_rows(q[:, 0:1], 1)
    ay = pyf - _rep_rows(q[:, 1:2], 1)
    az = pzf - _rep_rows(q[:, 2:3], 1)
    r2 = ax * ax + ay * ay + az * az
    p5 = jnp.concatenate([ax, ay, az, r2, jnp.ones_like(ax)], axis=1)
    d2s = jnp.dot(p5, m5, preferred_element_type=jnp.float32)
    nwx = jnp.maximum(1.0 - jnp.sqrt(jnp.maximum(d2s, 0.0)), 0.0)

    cwr = _rep_rows(cw, _K * _G)
    prod480 = nwx * cwr

    kg_ids = lax.broadcasted_iota(jnp.int32, (_K * _G, _G), 0)
    gg_ids = lax.broadcasted_iota(jnp.int32, (_K * _G, _G), 1)
    et = (kg_ids % _G == gg_ids).astype(jnp.float32)
    agg = jnp.dot(prod480, et, preferred_element_type=jnp.float32)

    g_ids = lax.broadcasted_iota(jnp.int32, (_G, _C), 0)
    c_ids = lax.broadcasted_iota(jnp.int32, (_G, _C), 1)
    expand = (c_ids // _CPG == g_ids).astype(jnp.float32)
    agg_exp = jnp.dot(agg, expand, preferred_element_type=jnp.float32)

    prod = agg_exp * feats
    o_ref[...] = jnp.sum(prod.reshape(_BN, _H, _C), axis=1)


def _tc_main(s_feats, q_pts, gathered, s1, s2, W1, b1r,
             gamr, betr, W2, b2r, kp3):
    return pl.pallas_call(
        _main_body,
        grid=(_NB,),
        in_specs=[
            pl.BlockSpec((_BN, _C), lambda i: (i, 0)),
            pl.BlockSpec((_BN, 3), lambda i: (i, 0)),
            pl.BlockSpec((_R, _DI), lambda i: (i, 0)),
            pl.BlockSpec((1, _CR), lambda i: (0, 0)),
            pl.BlockSpec((1, _CR), lambda i: (0, 0)),
            pl.BlockSpec((_C, _CR), lambda i: (0, 0)),
            pl.BlockSpec((1, _CR), lambda i: (0, 0)),
            pl.BlockSpec((1, _CR), lambda i: (0, 0)),
            pl.BlockSpec((1, _CR), lambda i: (0, 0)),
            pl.BlockSpec((_CR, _K * _G), lambda i: (0, 0)),
            pl.BlockSpec((1, _K * _G), lambda i: (0, 0)),
            pl.BlockSpec((5, _K * _G), lambda i: (0, 0)),
        ],
        out_specs=pl.BlockSpec((_BN, _C), lambda i: (i, 0)),
        out_shape=jax.ShapeDtypeStruct((_N, _C), jnp.float32),
        compiler_params=pltpu.CompilerParams(
            dimension_semantics=("arbitrary",)),
    )(s_feats, q_pts, gathered, s1, s2, W1, b1r,
      gamr, betr, W2, b2r, kp3)


def kernel(q_pts, s_pts, s_feats, neighb_inds, W1, b1, gamma, beta, W2, b2):
    hi = s_pts.astype(jnp.bfloat16)
    lo = (s_pts - hi.astype(jnp.float32)).astype(jnp.bfloat16)
    pos6 = jnp.stack([hi[:, 0], lo[:, 0], hi[:, 1], lo[:, 1],
                      hi[:, 2], lo[:, 2]], axis=1)
    meta = jnp.concatenate(
        [pos6, jnp.zeros((_N, _DI - 6), jnp.bfloat16)], axis=1)
    fw = jax.lax.bitcast_convert_type(
        s_feats.astype(jnp.bfloat16), jnp.uint16).astype(jnp.uint32)
    mww = jax.lax.bitcast_convert_type(
        meta, jnp.uint16).astype(jnp.uint32)
    table_i32 = jax.lax.bitcast_convert_type(
        fw | (mww << 16), jnp.int32)
    idx = neighb_inds.reshape(-1).astype(jnp.int32)
    idx = jnp.concatenate([idx, jnp.zeros((_BPAD - _B,), jnp.int32)])
    gathered = _sc_gather(table_i32, idx)
    b1r = b1.reshape(1, _CR)
    s1, s2 = _tc_stats(s_feats, W1, b1r)
    kpr = np.repeat(_kp_const().T, _G, axis=1)
    inv_s2 = 1.0 / (_SIGMA * _SIGMA)
    m5 = np.concatenate([
        -2.0 * inv_s2 * kpr,
        np.full((1, _K * _G), inv_s2, np.float32),
        inv_s2 * np.sum(kpr * kpr, axis=0, keepdims=True),
    ], axis=0).astype(np.float32)
    kp3 = jnp.asarray(m5)
    out = _tc_main(s_feats, q_pts, gathered, s1, s2, W1, b1r,
                   gamma.reshape(1, _CR), beta.reshape(1, _CR),
                   W2, b2.reshape(1, _K * _G), kp3)
    return out


# speedup vs baseline: 1.3391x; 1.3391x over previous
import functools

import jax
import jax.numpy as jnp
import numpy as np
from jax import lax
from jax.experimental import pallas as pl
from jax.experimental.pallas import tpu as pltpu
from jax.experimental.pallas import tpu_sc as plsc

_N = 10000
_H = 16
_C = 256
_K = 15
_CPG = 8
_G = _C // _CPG
_CR = 64
_SIGMA = 1.2
_BN_EPS = 1e-5

_DI = 256
_B = _N * _H
_NW = 32
_CHUNK = 40
_NITER = 125
_NCH = _NW * _NITER
_BPAD = _NCH * _CHUNK
_BOUT = _BPAD

_BN = 200
_NB = _N // _BN
_R = _BN * _H
_PR = _R // _CHUNK


def _kp_const():
    rng = np.random.RandomState(42)
    pts = rng.randn(_K, 3)
    pts = pts / (np.linalg.norm(pts, axis=1, keepdims=True) + 1e-9)
    r = rng.rand(_K, 1) ** (1.0 / 3.0)
    pts = pts * r * 1.2
    pts[0, :] = 0.0
    return pts.astype(np.float32)


def _sc_gather(table, idx):
    mesh = plsc.VectorSubcoreMesh(core_axis_name="c", subcore_axis_name="s")
    ipw = _NITER * _CHUNK

    @functools.partial(
        pl.kernel,
        mesh=mesh,
        out_type=jax.ShapeDtypeStruct((_BOUT, _DI), jnp.int32),
        scratch_types=[
            pltpu.VMEM((ipw,), jnp.int32),
            pltpu.VMEM((_CHUNK, _DI), jnp.int32),
            pltpu.VMEM((_CHUNK, _DI), jnp.int32),
            pltpu.SemaphoreType.DMA,
            pltpu.SemaphoreType.DMA,
            pltpu.SemaphoreType.DMA,
        ],
    )
    def k(table_hbm, idx_hbm, out_f,
          idx_v, rows0, rows1, gsem, osem0, osem1):
        wid = lax.axis_index("s") * 2 + lax.axis_index("c")
        base_chunk = wid * _NITER

        rows_v = (rows0, rows1)
        osem = (osem0, osem1)

        pltpu.sync_copy(idx_hbm.at[pl.ds(base_chunk * _CHUNK, ipw)], idx_v)

        pend_o = [None, None]
        for i in range(_NITER):
            b = i % 2
            ci = base_chunk + i
            if pend_o[b] is not None:
                pend_o[b].wait()
            pltpu.async_copy(
                table_hbm.at[idx_v.at[pl.ds(i * _CHUNK, _CHUNK)]],
                rows_v[b], gsem).wait()
            pend_o[b] = pltpu.async_copy(
                rows_v[b],
                out_f.at[pl.ds(ci * _CHUNK, _CHUNK)],
                osem[b])
        pend_o[0].wait()
        pend_o[1].wait()

    return k(table, idx)


def _stats_body(x_ref, w1_ref, b1_ref, s1_ref, s2_ref):
    h = jnp.dot(x_ref[...], w1_ref[...],
                preferred_element_type=jnp.float32) + b1_ref[...]

    @pl.when(pl.program_id(0) == 0)
    def _():
        s1_ref[...] = jnp.zeros_like(s1_ref)
        s2_ref[...] = jnp.zeros_like(s2_ref)

    s1_ref[...] += jnp.sum(h, axis=0, keepdims=True)
    s2_ref[...] += jnp.sum(h * h, axis=0, keepdims=True)


def _tc_stats(s_feats, W1, b1r):
    return pl.pallas_call(
        _stats_body,
        grid=(_NB,),
        in_specs=[
            pl.BlockSpec((_BN, _C), lambda i: (i, 0)),
            pl.BlockSpec((_C, _CR), lambda i: (0, 0)),
            pl.BlockSpec((1, _CR), lambda i: (0, 0)),
        ],
        out_specs=[
            pl.BlockSpec((1, _CR), lambda i: (0, 0)),
            pl.BlockSpec((1, _CR), lambda i: (0, 0)),
        ],
        out_shape=[
            jax.ShapeDtypeStruct((1, _CR), jnp.float32),
            jax.ShapeDtypeStruct((1, _CR), jnp.float32),
        ],
        compiler_params=pltpu.CompilerParams(
            dimension_semantics=("arbitrary",)),
    )(s_feats, W1, b1r)


def _rep_rows(a, m):
    return jnp.broadcast_to(a[:, None, :], (_BN, _H, m)).reshape(_R, m)


def _main_body(x_ref, q_ref, g_ref,
               s1_ref, s2_ref, w1_ref, b1_ref,
               gam_ref, bet_ref, w2_ref, b2_ref, kp_ref, o_ref):
    x = x_ref[...]
    h = jnp.dot(x, w1_ref[...], preferred_element_type=jnp.float32)
    h = h + b1_ref[...]
    mean = s1_ref[...] * (1.0 / _N)
    var = s2_ref[...] * (1.0 / _N) - mean * mean
    inv = lax.rsqrt(var + _BN_EPS)
    h = (h - mean) * (inv * gam_ref[...]) + bet_ref[...]
    h = jnp.where(h >= 0.0, h, 0.1 * h)
    cw = jnp.dot(h, w2_ref[...], preferred_element_type=jnp.float32)
    cw = cw + b2_ref[...]

    gw = g_ref[...]
    feats = jax.lax.bitcast_convert_type(
        jax.lax.shift_left(gw, 16), jnp.float32)
    q = q_ref[...]
    m5 = kp_ref[...]

    mw = gw[:, 0:6]
    pos = jax.lax.bitcast_convert_type(
        jax.lax.bitwise_and(mw, jnp.int32(-65536)), jnp.float32)
    pxf = pos[:, 0:1] + pos[:, 1:2]
    pyf = pos[:, 2:3] + pos[:, 3:4]
    pzf = pos[:, 4:5] + pos[:, 5:6]
    ax = pxf - _rep_rows(q[:, 0:1], 1)
    ay = pyf - _rep_rows(q[:, 1:2], 1)
    az = pzf - _rep_rows(q[:, 2:3], 1)
    r2 = ax * ax + ay * ay + az * az
    p5 = jnp.concatenate([ax, ay, az, r2, jnp.ones_like(ax)], axis=1)
    d2s = jnp.dot(p5, m5, preferred_element_type=jnp.float32)
    nwx = jnp.maximum(1.0 - jnp.sqrt(jnp.maximum(d2s, 0.0)), 0.0)

    cwr = _rep_rows(cw, _K * _G)
    prod480 = nwx * cwr

    kg_ids = lax.broadcasted_iota(jnp.int32, (_K * _G, _G), 0)
    gg_ids = lax.broadcasted_iota(jnp.int32, (_K * _G, _G), 1)
    et = (kg_ids % _G == gg_ids).astype(jnp.float32)
    agg = jnp.dot(prod480, et, preferred_element_type=jnp.float32)

    g_ids = lax.broadcasted_iota(jnp.int32, (_G, _C), 0)
    c_ids = lax.broadcasted_iota(jnp.int32, (_G, _C), 1)
    expand = (c_ids // _CPG == g_ids).astype(jnp.float32)
    agg_exp = jnp.dot(agg, expand, preferred_element_type=jnp.float32)

    prod = agg_exp * feats
    o_ref[...] = jnp.sum(prod.reshape(_BN, _H, _C), axis=1)


def _tc_main(s_feats, q_pts, gathered, s1, s2, W1, b1r,
             gamr, betr, W2, b2r, kp3):
    return pl.pallas_call(
        _main_body,
        grid=(_NB,),
        in_specs=[
            pl.BlockSpec((_BN, _C), lambda i: (i, 0)),
            pl.BlockSpec((_BN, 3), lambda i: (i, 0)),
            pl.BlockSpec((_R, _DI), lambda i: (i, 0)),
            pl.BlockSpec((1, _CR), lambda i: (0, 0)),
            pl.BlockSpec((1, _CR), lambda i: (0, 0)),
            pl.BlockSpec((_C, _CR), lambda i: (0, 0)),
            pl.BlockSpec((1, _CR), lambda i: (0, 0)),
            pl.BlockSpec((1, _CR), lambda i: (0, 0)),
            pl.BlockSpec((1, _CR), lambda i: (0, 0)),
            pl.BlockSpec((_CR, _K * _G), lambda i: (0, 0)),
            pl.BlockSpec((1, _K * _G), lambda i: (0, 0)),
            pl.BlockSpec((5, _K * _G), lambda i: (0, 0)),
        ],
        out_specs=pl.BlockSpec((_BN, _C), lambda i: (i, 0)),
        out_shape=jax.ShapeDtypeStruct((_N, _C), jnp.float32),
        compiler_params=pltpu.CompilerParams(
            dimension_semantics=("arbitrary",)),
    )(s_feats, q_pts, gathered, s1, s2, W1, b1r,
      gamr, betr, W2, b2r, kp3)


def kernel(q_pts, s_pts, s_feats, neighb_inds, W1, b1, gamma, beta, W2, b2):
    hi = s_pts.astype(jnp.bfloat16)
    lo = (s_pts - hi.astype(jnp.float32)).astype(jnp.bfloat16)
    pos6 = jnp.stack([hi[:, 0], lo[:, 0], hi[:, 1], lo[:, 1],
                      hi[:, 2], lo[:, 2]], axis=1)
    meta = jnp.concatenate(
        [pos6, jnp.zeros((_N, _DI - 6), jnp.bfloat16)], axis=1)
    fw = jax.lax.bitcast_convert_type(
        s_feats.astype(jnp.bfloat16), jnp.uint16).astype(jnp.uint32)
    mww = jax.lax.bitcast_convert_type(
        meta, jnp.uint16).astype(jnp.uint32)
    table_i32 = jax.lax.bitcast_convert_type(
        fw | (mww << 16), jnp.int32)
    idx = neighb_inds.reshape(-1).astype(jnp.int32)
    idx = jnp.concatenate([idx, jnp.zeros((_BPAD - _B,), jnp.int32)])
    gathered = _sc_gather(table_i32, idx)
    b1r = b1.reshape(1, _CR)
    s1, s2 = _tc_stats(s_feats, W1, b1r)
    kpr = np.repeat(_kp_const().T, _G, axis=1)
    inv_s2 = 1.0 / (_SIGMA * _SIGMA)
    m5 = np.concatenate([
        -2.0 * inv_s2 * kpr,
        np.full((1, _K * _G), inv_s2, np.float32),
        inv_s2 * np.sum(kpr * kpr, axis=0, keepdims=True),
    ], axis=0).astype(np.float32)
    kp3 = jnp.asarray(m5)
    out = _tc_main(s_feats, q_pts, gathered, s1, s2, W1, b1r,
                   gamma.reshape(1, _CR), beta.reshape(1, _CR),
                   W2, b2.reshape(1, _K * _G), kp3)
    return out


# speedup vs baseline: 1.3940x; 1.0411x over previous
import functools

import jax
import jax.numpy as jnp
import numpy as np
from jax import lax
from jax.experimental import pallas as pl
from jax.experimental.pallas import tpu as pltpu
from jax.experimental.pallas import tpu_sc as plsc

_N = 10000
_H = 16
_C = 256
_K = 15
_CPG = 8
_G = _C // _CPG
_CR = 64
_SIGMA = 1.2
_BN_EPS = 1e-5

_DI = 256
_B = _N * _H
_NW = 32
_CHUNK = 40
_NITER = 125
_NCH = _NW * _NITER
_BPAD = _NCH * _CHUNK
_BOUT = _BPAD

_BN = 200
_NB = _N // _BN
_R = _BN * _H
_PR = _R // _CHUNK


def _kp_const():
    rng = np.random.RandomState(42)
    pts = rng.randn(_K, 3)
    pts = pts / (np.linalg.norm(pts, axis=1, keepdims=True) + 1e-9)
    r = rng.rand(_K, 1) ** (1.0 / 3.0)
    pts = pts * r * 1.2
    pts[0, :] = 0.0
    return pts.astype(np.float32)


def _sc_gather(table, idx):
    mesh = plsc.VectorSubcoreMesh(core_axis_name="c", subcore_axis_name="s")
    ipw = _NITER * _CHUNK

    @functools.partial(
        pl.kernel,
        mesh=mesh,
        out_type=jax.ShapeDtypeStruct((_BOUT, _DI), jnp.int32),
        scratch_types=[
            pltpu.VMEM((ipw,), jnp.int32),
            pltpu.VMEM((_CHUNK, _DI), jnp.int32),
            pltpu.VMEM((_CHUNK, _DI), jnp.int32),
            pltpu.SemaphoreType.DMA,
            pltpu.SemaphoreType.DMA,
            pltpu.SemaphoreType.DMA,
        ],
    )
    def k(table_hbm, idx_hbm, out_f,
          idx_v, rows0, rows1, gsem, osem0, osem1):
        wid = lax.axis_index("s") * 2 + lax.axis_index("c")
        base_chunk = wid * _NITER

        rows_v = (rows0, rows1)
        osem = (osem0, osem1)

        pltpu.sync_copy(idx_hbm.at[pl.ds(base_chunk * _CHUNK, ipw)], idx_v)

        pend_o = [None, None]
        for i in range(_NITER):
            b = i % 2
            ci = base_chunk + i
            if pend_o[b] is not None:
                pend_o[b].wait()
            pltpu.async_copy(
                table_hbm.at[idx_v.at[pl.ds(i * _CHUNK, _CHUNK)]],
                rows_v[b], gsem).wait()
            pend_o[b] = pltpu.async_copy(
                rows_v[b],
                out_f.at[pl.ds(ci * _CHUNK, _CHUNK)],
                osem[b])
        pend_o[0].wait()
        pend_o[1].wait()

    return k(table, idx)


def _stats_body(x_ref, w1_ref, b1_ref, s1_ref, s2_ref):
    h = jnp.dot(x_ref[...], w1_ref[...],
                preferred_element_type=jnp.float32) + b1_ref[...]

    @pl.when(pl.program_id(0) == 0)
    def _():
        s1_ref[...] = jnp.zeros_like(s1_ref)
        s2_ref[...] = jnp.zeros_like(s2_ref)

    s1_ref[...] += jnp.sum(h, axis=0, keepdims=True)
    s2_ref[...] += jnp.sum(h * h, axis=0, keepdims=True)


def _tc_stats(s_feats, W1, b1r):
    return pl.pallas_call(
        _stats_body,
        grid=(_NB,),
        in_specs=[
            pl.BlockSpec((_BN, _C), lambda i: (i, 0)),
            pl.BlockSpec((_C, _CR), lambda i: (0, 0)),
            pl.BlockSpec((1, _CR), lambda i: (0, 0)),
        ],
        out_specs=[
            pl.BlockSpec((1, _CR), lambda i: (0, 0)),
            pl.BlockSpec((1, _CR), lambda i: (0, 0)),
        ],
        out_shape=[
            jax.ShapeDtypeStruct((1, _CR), jnp.float32),
            jax.ShapeDtypeStruct((1, _CR), jnp.float32),
        ],
        compiler_params=pltpu.CompilerParams(
            dimension_semantics=("arbitrary",)),
    )(s_feats, W1, b1r)


def _rep_rows(a, m):
    return jnp.broadcast_to(a[:, None, :], (_BN, _H, m)).reshape(_R, m)


def _main_body(x_ref, q_ref, g_ref,
               s1_ref, s2_ref, w1_ref, b1_ref,
               gam_ref, bet_ref, w2_ref, b2_ref, kp_ref, o_ref):
    x = x_ref[...]
    h = jnp.dot(x, w1_ref[...], preferred_element_type=jnp.float32)
    h = h + b1_ref[...]
    mean = s1_ref[...] * (1.0 / _N)
    var = s2_ref[...] * (1.0 / _N) - mean * mean
    inv = lax.rsqrt(var + _BN_EPS)
    h = (h - mean) * (inv * gam_ref[...]) + bet_ref[...]
    h = jnp.where(h >= 0.0, h, 0.1 * h)
    cw = jnp.dot(h, w2_ref[...], preferred_element_type=jnp.float32)
    cw = cw + b2_ref[...]

    gw = g_ref[...]
    feats = jax.lax.bitcast_convert_type(
        jax.lax.shift_left(gw, 16), jnp.float32)
    q = q_ref[...]
    m5 = kp_ref[...]

    mw = gw[:, 0:6]
    pos = jax.lax.bitcast_convert_type(
        jax.lax.bitwise_and(mw, jnp.int32(-65536)), jnp.float32)
    pxf = pos[:, 0:1] + pos[:, 1:2]
    pyf = pos[:, 2:3] + pos[:, 3:4]
    pzf = pos[:, 4:5] + pos[:, 5:6]
    ax = pxf - _rep_rows(q[:, 0:1], 1)
    ay = pyf - _rep_rows(q[:, 1:2], 1)
    az = pzf - _rep_rows(q[:, 2:3], 1)
    r2 = ax * ax + ay * ay + az * az
    p5 = jnp.concatenate([ax, ay, az, r2, jnp.ones_like(ax)], axis=1)
    d2s = jnp.dot(p5, m5, preferred_element_type=jnp.float32)
    nws = jnp.maximum(1.0 - jnp.sqrt(jnp.maximum(d2s, 0.0)), 0.0)

    k_ids = lax.broadcasted_iota(jnp.int32, (_K, _K * _G), 0)
    col_ids = lax.broadcasted_iota(jnp.int32, (_K, _K * _G), 1)
    rep = (col_ids // _G == k_ids).astype(jnp.float32)
    nwx = jnp.dot(nws, rep, preferred_element_type=jnp.float32)

    cwr = _rep_rows(cw, _K * _G)
    prod480 = nwx * cwr

    kg_ids = lax.broadcasted_iota(jnp.int32, (_K * _G, _G), 0)
    gg_ids = lax.broadcasted_iota(jnp.int32, (_K * _G, _G), 1)
    et = (kg_ids % _G == gg_ids).astype(jnp.float32)
    agg = jnp.dot(prod480, et, preferred_element_type=jnp.float32)

    g_ids = lax.broadcasted_iota(jnp.int32, (_G, _C), 0)
    c_ids = lax.broadcasted_iota(jnp.int32, (_G, _C), 1)
    expand = (c_ids // _CPG == g_ids).astype(jnp.float32)
    agg_exp = jnp.dot(agg, expand, preferred_element_type=jnp.float32)

    prod = agg_exp * feats
    o_ref[...] = jnp.sum(prod.reshape(_BN, _H, _C), axis=1)


def _tc_main(s_feats, q_pts, gathered, s1, s2, W1, b1r,
             gamr, betr, W2, b2r, kp3):
    return pl.pallas_call(
        _main_body,
        grid=(_NB,),
        in_specs=[
            pl.BlockSpec((_BN, _C), lambda i: (i, 0)),
            pl.BlockSpec((_BN, 3), lambda i: (i, 0)),
            pl.BlockSpec((_R, _DI), lambda i: (i, 0)),
            pl.BlockSpec((1, _CR), lambda i: (0, 0)),
            pl.BlockSpec((1, _CR), lambda i: (0, 0)),
            pl.BlockSpec((_C, _CR), lambda i: (0, 0)),
            pl.BlockSpec((1, _CR), lambda i: (0, 0)),
            pl.BlockSpec((1, _CR), lambda i: (0, 0)),
            pl.BlockSpec((1, _CR), lambda i: (0, 0)),
            pl.BlockSpec((_CR, _K * _G), lambda i: (0, 0)),
            pl.BlockSpec((1, _K * _G), lambda i: (0, 0)),
            pl.BlockSpec((5, _K), lambda i: (0, 0)),
        ],
        out_specs=pl.BlockSpec((_BN, _C), lambda i: (i, 0)),
        out_shape=jax.ShapeDtypeStruct((_N, _C), jnp.float32),
        compiler_params=pltpu.CompilerParams(
            dimension_semantics=("arbitrary",)),
    )(s_feats, q_pts, gathered, s1, s2, W1, b1r,
      gamr, betr, W2, b2r, kp3)


def kernel(q_pts, s_pts, s_feats, neighb_inds, W1, b1, gamma, beta, W2, b2):
    hi = s_pts.astype(jnp.bfloat16)
    lo = (s_pts - hi.astype(jnp.float32)).astype(jnp.bfloat16)
    pos6 = jnp.stack([hi[:, 0], lo[:, 0], hi[:, 1], lo[:, 1],
                      hi[:, 2], lo[:, 2]], axis=1)
    meta = jnp.concatenate(
        [pos6, jnp.zeros((_N, _DI - 6), jnp.bfloat16)], axis=1)
    fw = jax.lax.bitcast_convert_type(
        s_feats.astype(jnp.bfloat16), jnp.uint16).astype(jnp.uint32)
    mww = jax.lax.bitcast_convert_type(
        meta, jnp.uint16).astype(jnp.uint32)
    table_i32 = jax.lax.bitcast_convert_type(
        fw | (mww << 16), jnp.int32)
    idx = neighb_inds.reshape(-1).astype(jnp.int32)
    idx = jnp.concatenate([idx, jnp.zeros((_BPAD - _B,), jnp.int32)])
    gathered = _sc_gather(table_i32, idx)
    b1r = b1.reshape(1, _CR)
    s1, s2 = _tc_stats(s_feats, W1, b1r)
    kpr = _kp_const().T
    inv_s2 = 1.0 / (_SIGMA * _SIGMA)
    m5 = np.concatenate([
        -2.0 * inv_s2 * kpr,
        np.full((1, _K), inv_s2, np.float32),
        inv_s2 * np.sum(kpr * kpr, axis=0, keepdims=True),
    ], axis=0).astype(np.float32)
    kp3 = jnp.asarray(m5)
    out = _tc_main(s_feats, q_pts, gathered, s1, s2, W1, b1r,
                   gamma.reshape(1, _CR), beta.reshape(1, _CR),
                   W2, b2.reshape(1, _K * _G), kp3)
    return out


# speedup vs baseline: 1.4156x; 1.0155x over previous
import functools

import jax
import jax.numpy as jnp
import numpy as np
from jax import lax
from jax.experimental import pallas as pl
from jax.experimental.pallas import tpu as pltpu
from jax.experimental.pallas import tpu_sc as plsc

_N = 10000
_H = 16
_C = 256
_K = 15
_CPG = 8
_G = _C // _CPG
_CR = 64
_SIGMA = 1.2
_BN_EPS = 1e-5

_DI = 256
_B = _N * _H
_NW = 32
_CHUNK = 40
_NITER = 125
_NCH = _NW * _NITER
_BPAD = _NCH * _CHUNK
_BOUT = _BPAD

_BN = 400
_NB = _N // _BN
_R = _BN * _H
_PR = _R // _CHUNK


def _kp_const():
    rng = np.random.RandomState(42)
    pts = rng.randn(_K, 3)
    pts = pts / (np.linalg.norm(pts, axis=1, keepdims=True) + 1e-9)
    r = rng.rand(_K, 1) ** (1.0 / 3.0)
    pts = pts * r * 1.2
    pts[0, :] = 0.0
    return pts.astype(np.float32)


def _sc_gather(table, idx):
    mesh = plsc.VectorSubcoreMesh(core_axis_name="c", subcore_axis_name="s")
    ipw = _NITER * _CHUNK

    @functools.partial(
        pl.kernel,
        mesh=mesh,
        out_type=jax.ShapeDtypeStruct((_BOUT, _DI), jnp.int32),
        scratch_types=[
            pltpu.VMEM((ipw,), jnp.int32),
            pltpu.VMEM((_CHUNK, _DI), jnp.int32),
            pltpu.VMEM((_CHUNK, _DI), jnp.int32),
            pltpu.SemaphoreType.DMA,
            pltpu.SemaphoreType.DMA,
            pltpu.SemaphoreType.DMA,
        ],
    )
    def k(table_hbm, idx_hbm, out_f,
          idx_v, rows0, rows1, gsem, osem0, osem1):
        wid = lax.axis_index("s") * 2 + lax.axis_index("c")
        base_chunk = wid * _NITER

        rows_v = (rows0, rows1)
        osem = (osem0, osem1)

        pltpu.sync_copy(idx_hbm.at[pl.ds(base_chunk * _CHUNK, ipw)], idx_v)

        pend_o = [None, None]
        for i in range(_NITER):
            b = i % 2
            ci = base_chunk + i
            if pend_o[b] is not None:
                pend_o[b].wait()
            pltpu.async_copy(
                table_hbm.at[idx_v.at[pl.ds(i * _CHUNK, _CHUNK)]],
                rows_v[b], gsem).wait()
            pend_o[b] = pltpu.async_copy(
                rows_v[b],
                out_f.at[pl.ds(ci * _CHUNK, _CHUNK)],
                osem[b])
        pend_o[0].wait()
        pend_o[1].wait()

    return k(table, idx)


def _stats_body(x_ref, w1_ref, b1_ref, s1_ref, s2_ref):
    h = jnp.dot(x_ref[...], w1_ref[...],
                preferred_element_type=jnp.float32) + b1_ref[...]

    @pl.when(pl.program_id(0) == 0)
    def _():
        s1_ref[...] = jnp.zeros_like(s1_ref)
        s2_ref[...] = jnp.zeros_like(s2_ref)

    s1_ref[...] += jnp.sum(h, axis=0, keepdims=True)
    s2_ref[...] += jnp.sum(h * h, axis=0, keepdims=True)


def _tc_stats(s_feats, W1, b1r):
    return pl.pallas_call(
        _stats_body,
        grid=(_NB,),
        in_specs=[
            pl.BlockSpec((_BN, _C), lambda i: (i, 0)),
            pl.BlockSpec((_C, _CR), lambda i: (0, 0)),
            pl.BlockSpec((1, _CR), lambda i: (0, 0)),
        ],
        out_specs=[
            pl.BlockSpec((1, _CR), lambda i: (0, 0)),
            pl.BlockSpec((1, _CR), lambda i: (0, 0)),
        ],
        out_shape=[
            jax.ShapeDtypeStruct((1, _CR), jnp.float32),
            jax.ShapeDtypeStruct((1, _CR), jnp.float32),
        ],
        compiler_params=pltpu.CompilerParams(
            dimension_semantics=("arbitrary",)),
    )(s_feats, W1, b1r)


def _rep_rows(a, m):
    return jnp.broadcast_to(a[:, None, :], (_BN, _H, m)).reshape(_R, m)


def _main_body(x_ref, q_ref, g_ref,
               s1_ref, s2_ref, w1_ref, b1_ref,
               gam_ref, bet_ref, w2_ref, b2_ref, kp_ref, o_ref):
    x = x_ref[...]
    h = jnp.dot(x, w1_ref[...], preferred_element_type=jnp.float32)
    h = h + b1_ref[...]
    mean = s1_ref[...] * (1.0 / _N)
    var = s2_ref[...] * (1.0 / _N) - mean * mean
    inv = lax.rsqrt(var + _BN_EPS)
    h = (h - mean) * (inv * gam_ref[...]) + bet_ref[...]
    h = jnp.where(h >= 0.0, h, 0.1 * h)
    cw = jnp.dot(h, w2_ref[...], preferred_element_type=jnp.float32)
    cw = cw + b2_ref[...]

    gw = g_ref[...]
    feats = jax.lax.bitcast_convert_type(
        jax.lax.shift_left(gw, 16), jnp.float32)
    q = q_ref[...]
    m5 = kp_ref[...]

    mw = gw[:, 0:6]
    pos = jax.lax.bitcast_convert_type(
        jax.lax.bitwise_and(mw, jnp.int32(-65536)), jnp.float32)
    pxf = pos[:, 0:1] + pos[:, 1:2]
    pyf = pos[:, 2:3] + pos[:, 3:4]
    pzf = pos[:, 4:5] + pos[:, 5:6]
    ax = pxf - _rep_rows(q[:, 0:1], 1)
    ay = pyf - _rep_rows(q[:, 1:2], 1)
    az = pzf - _rep_rows(q[:, 2:3], 1)
    r2 = ax * ax + ay * ay + az * az
    p5 = jnp.concatenate([ax, ay, az, r2, jnp.ones_like(ax)], axis=1)
    d2s = jnp.dot(p5, m5, preferred_element_type=jnp.float32)
    nws = jnp.maximum(1.0 - jnp.sqrt(jnp.maximum(d2s, 0.0)), 0.0)

    k_ids = lax.broadcasted_iota(jnp.int32, (_K, _K * _G), 0)
    col_ids = lax.broadcasted_iota(jnp.int32, (_K, _K * _G), 1)
    rep = (col_ids // _G == k_ids).astype(jnp.float32)
    nwx = jnp.dot(nws, rep, preferred_element_type=jnp.float32)

    cwr = _rep_rows(cw, _K * _G)
    prod480 = nwx * cwr

    kg_ids = lax.broadcasted_iota(jnp.int32, (_K * _G, _G), 0)
    gg_ids = lax.broadcasted_iota(jnp.int32, (_K * _G, _G), 1)
    et = (kg_ids % _G == gg_ids).astype(jnp.float32)
    agg = jnp.dot(prod480, et, preferred_element_type=jnp.float32)

    g_ids = lax.broadcasted_iota(jnp.int32, (_G, _C), 0)
    c_ids = lax.broadcasted_iota(jnp.int32, (_G, _C), 1)
    expand = (c_ids // _CPG == g_ids).astype(jnp.float32)
    agg_exp = jnp.dot(agg, expand, preferred_element_type=jnp.float32)

    prod = agg_exp * feats
    o_ref[...] = jnp.sum(prod.reshape(_BN, _H, _C), axis=1)


def _tc_main(s_feats, q_pts, gathered, s1, s2, W1, b1r,
             gamr, betr, W2, b2r, kp3):
    return pl.pallas_call(
        _main_body,
        grid=(_NB,),
        in_specs=[
            pl.BlockSpec((_BN, _C), lambda i: (i, 0)),
            pl.BlockSpec((_BN, 3), lambda i: (i, 0)),
            pl.BlockSpec((_R, _DI), lambda i: (i, 0)),
            pl.BlockSpec((1, _CR), lambda i: (0, 0)),
            pl.BlockSpec((1, _CR), lambda i: (0, 0)),
            pl.BlockSpec((_C, _CR), lambda i: (0, 0)),
            pl.BlockSpec((1, _CR), lambda i: (0, 0)),
            pl.BlockSpec((1, _CR), lambda i: (0, 0)),
            pl.BlockSpec((1, _CR), lambda i: (0, 0)),
            pl.BlockSpec((_CR, _K * _G), lambda i: (0, 0)),
            pl.BlockSpec((1, _K * _G), lambda i: (0, 0)),
            pl.BlockSpec((5, _K), lambda i: (0, 0)),
        ],
        out_specs=pl.BlockSpec((_BN, _C), lambda i: (i, 0)),
        out_shape=jax.ShapeDtypeStruct((_N, _C), jnp.float32),
        compiler_params=pltpu.CompilerParams(
            dimension_semantics=("arbitrary",)),
    )(s_feats, q_pts, gathered, s1, s2, W1, b1r,
      gamr, betr, W2, b2r, kp3)


def kernel(q_pts, s_pts, s_feats, neighb_inds, W1, b1, gamma, beta, W2, b2):
    hi = s_pts.astype(jnp.bfloat16)
    lo = (s_pts - hi.astype(jnp.float32)).astype(jnp.bfloat16)
    pos6 = jnp.stack([hi[:, 0], lo[:, 0], hi[:, 1], lo[:, 1],
                      hi[:, 2], lo[:, 2]], axis=1)
    meta = jnp.concatenate(
        [pos6, jnp.zeros((_N, _DI - 6), jnp.bfloat16)], axis=1)
    fw = jax.lax.bitcast_convert_type(
        s_feats.astype(jnp.bfloat16), jnp.uint16).astype(jnp.uint32)
    mww = jax.lax.bitcast_convert_type(
        meta, jnp.uint16).astype(jnp.uint32)
    table_i32 = jax.lax.bitcast_convert_type(
        fw | (mww << 16), jnp.int32)
    idx = neighb_inds.reshape(-1).astype(jnp.int32)
    idx = jnp.concatenate([idx, jnp.zeros((_BPAD - _B,), jnp.int32)])
    gathered = _sc_gather(table_i32, idx)
    b1r = b1.reshape(1, _CR)
    s1, s2 = _tc_stats(s_feats, W1, b1r)
    kpr = _kp_const().T
    inv_s2 = 1.0 / (_SIGMA * _SIGMA)
    m5 = np.concatenate([
        -2.0 * inv_s2 * kpr,
        np.full((1, _K), inv_s2, np.float32),
        inv_s2 * np.sum(kpr * kpr, axis=0, keepdims=True),
    ], axis=0).astype(np.float32)
    kp3 = jnp.asarray(m5)
    out = _tc_main(s_feats, q_pts, gathered, s1, s2, W1, b1r,
                   gamma.reshape(1, _CR), beta.reshape(1, _CR),
                   W2, b2.reshape(1, _K * _G), kp3)
    return out
